# baseline (device time: 432581 ns/iter reference)
import jax
import jax.numpy as jnp
from jax import lax
from jax.experimental import pallas as pl
from jax.experimental.pallas import tpu as pltpu

M = 32768
N = 1024
H = M // 2

CHS = [128, 128, 256, 512] + [1024] * 14 + [512, 256, 128, 128]
assert sum(CHS) == H
NC = len(CHS)
OFFS = [sum(CHS[:c]) for c in range(NC)]
CHMAX = max(CHS)


def kernel(x):
    def body(
        x_ref,
        out_ref,
        a_f32,
        a_bf16,
        xrecv,
        in_sems,
        st_sems,
        xsend_sems,
        xrecv_sems,
        ysend_sems,
        yrecv_sems,
    ):
        my_x = lax.axis_index("x")
        my_y = lax.axis_index("y")
        x_nbr = (1 - my_x, my_y)
        y_nbr = (my_x, 1 - my_y)
        row0 = my_y * H
        other0 = (1 - my_y) * H

        def copy_in(c):
            return pltpu.make_async_copy(
                x_ref.at[pl.ds(row0 + OFFS[c], CHS[c])],
                a_f32.at[c % 2, pl.ds(0, CHS[c])],
                in_sems.at[c],
            )

        def x_rdma(c):
            return pltpu.make_async_remote_copy(
                src_ref=a_bf16.at[c % 2, pl.ds(0, CHS[c])],
                dst_ref=xrecv.at[pl.ds(OFFS[c], CHS[c])],
                send_sem=xsend_sems.at[c],
                recv_sem=xrecv_sems.at[c],
                device_id=x_nbr,
                device_id_type=pl.DeviceIdType.MESH,
            )

        def y_rdma_send(c):
            return pltpu.make_async_remote_copy(
                src_ref=xrecv.at[pl.ds(OFFS[c], CHS[c])],
                dst_ref=out_ref.at[pl.ds(row0 + OFFS[c], CHS[c])],
                send_sem=ysend_sems.at[c],
                recv_sem=yrecv_sems.at[c],
                device_id=y_nbr,
                device_id_type=pl.DeviceIdType.MESH,
            )

        def y_rdma_recv(c):
            return pltpu.make_async_remote_copy(
                src_ref=xrecv.at[pl.ds(OFFS[c], CHS[c])],
                dst_ref=out_ref.at[pl.ds(other0 + OFFS[c], CHS[c])],
                send_sem=ysend_sems.at[c],
                recv_sem=yrecv_sems.at[c],
                device_id=y_nbr,
                device_id_type=pl.DeviceIdType.MESH,
            )

        def st_copy(c):
            return pltpu.make_async_copy(
                xrecv.at[pl.ds(OFFS[c], CHS[c])],
                out_ref.at[pl.ds(row0 + OFFS[c], CHS[c])],
                st_sems.at[c],
            )

        copy_in(0).start()
        copy_in(1).start()

        barrier = pltpu.get_barrier_semaphore()
        for nbr in (x_nbr, y_nbr):
            pl.semaphore_signal(
                barrier, inc=1, device_id=nbr, device_id_type=pl.DeviceIdType.MESH
            )
        pl.semaphore_wait(barrier, 2)

        def issue_send(c):
            copy_in(c).wait()
            if c >= 2:
                x_rdma(c - 2).wait_send()
            a_bf16[c % 2, 0 : CHS[c], :] = a_f32[c % 2, 0 : CHS[c], :].astype(
                jnp.bfloat16
            )
            if c + 2 < NC:
                copy_in(c + 2).start()
            x_rdma(c).start()

        issue_send(0)
        issue_send(1)

        for c in range(NC):
            x_rdma(c).wait_recv()
            lo, hi = OFFS[c], OFFS[c] + CHS[c]
            xrecv[lo:hi, :] = xrecv[lo:hi, :] + a_bf16[c % 2, 0 : CHS[c], :]

            st_copy(c).start()
            y_rdma_send(c).start()
            if c + 2 < NC:
                issue_send(c + 2)

        for c in range(NC):
            y_rdma_recv(c).wait_recv()

        for c in (NC - 2, NC - 1):
            x_rdma(c).wait_send()
        for c in range(NC):
            y_rdma_send(c).wait_send()
            st_copy(c).wait()

    return pl.pallas_call(
        body,
        out_shape=jax.ShapeDtypeStruct((M, N), jnp.bfloat16),
        in_specs=[pl.BlockSpec(memory_space=pl.ANY)],
        out_specs=pl.BlockSpec(memory_space=pl.ANY),
        scratch_shapes=[
            pltpu.VMEM((2, CHMAX, N), jnp.float32),
            pltpu.VMEM((2, CHMAX, N), jnp.bfloat16),
            pltpu.VMEM((H, N), jnp.bfloat16),
            pltpu.SemaphoreType.DMA((NC,)),
            pltpu.SemaphoreType.DMA((NC,)),
            pltpu.SemaphoreType.DMA((NC,)),
            pltpu.SemaphoreType.DMA((NC,)),
            pltpu.SemaphoreType.DMA((NC,)),
            pltpu.SemaphoreType.DMA((NC,)),
        ],
        compiler_params=pltpu.CompilerParams(
            collective_id=0, vmem_limit_bytes=96 * 1024 * 1024
        ),
    )(x)


# device time: 421904 ns/iter; 1.0253x vs baseline; 1.0253x over previous
import jax
import jax.numpy as jnp
from jax import lax
from jax.experimental import pallas as pl
from jax.experimental.pallas import tpu as pltpu

M = 32768
N = 1024
H = M // 2

CHS = [128, 128, 256] + [512] * 30 + [256, 128, 128]
assert sum(CHS) == H
NC = len(CHS)
OFFS = [sum(CHS[:c]) for c in range(NC)]
CHMAX = max(CHS)


def kernel(x):
    def body(
        x_ref,
        out_ref,
        a_f32,
        a_bf16,
        xrecv,
        in_sems,
        st_sems,
        xsend_sems,
        xrecv_sems,
        ysend_sems,
        yrecv_sems,
    ):
        my_x = lax.axis_index("x")
        my_y = lax.axis_index("y")
        x_nbr = (1 - my_x, my_y)
        y_nbr = (my_x, 1 - my_y)
        row0 = my_y * H
        other0 = (1 - my_y) * H

        def copy_in(c):
            return pltpu.make_async_copy(
                x_ref.at[pl.ds(row0 + OFFS[c], CHS[c])],
                a_f32.at[c % 2, pl.ds(0, CHS[c])],
                in_sems.at[c],
            )

        def x_rdma(c):
            return pltpu.make_async_remote_copy(
                src_ref=a_bf16.at[c % 2, pl.ds(0, CHS[c])],
                dst_ref=xrecv.at[pl.ds(OFFS[c], CHS[c])],
                send_sem=xsend_sems.at[c],
                recv_sem=xrecv_sems.at[c],
                device_id=x_nbr,
                device_id_type=pl.DeviceIdType.MESH,
            )

        def y_rdma_send(c):
            return pltpu.make_async_remote_copy(
                src_ref=xrecv.at[pl.ds(OFFS[c], CHS[c])],
                dst_ref=out_ref.at[pl.ds(row0 + OFFS[c], CHS[c])],
                send_sem=ysend_sems.at[c],
                recv_sem=yrecv_sems.at[c],
                device_id=y_nbr,
                device_id_type=pl.DeviceIdType.MESH,
            )

        def y_rdma_recv(c):
            return pltpu.make_async_remote_copy(
                src_ref=xrecv.at[pl.ds(OFFS[c], CHS[c])],
                dst_ref=out_ref.at[pl.ds(other0 + OFFS[c], CHS[c])],
                send_sem=ysend_sems.at[c],
                recv_sem=yrecv_sems.at[c],
                device_id=y_nbr,
                device_id_type=pl.DeviceIdType.MESH,
            )

        def st_copy(c):
            return pltpu.make_async_copy(
                xrecv.at[pl.ds(OFFS[c], CHS[c])],
                out_ref.at[pl.ds(row0 + OFFS[c], CHS[c])],
                st_sems.at[c],
            )

        copy_in(0).start()
        copy_in(1).start()

        barrier = pltpu.get_barrier_semaphore()
        for nbr in (x_nbr, y_nbr):
            pl.semaphore_signal(
                barrier, inc=1, device_id=nbr, device_id_type=pl.DeviceIdType.MESH
            )
        pl.semaphore_wait(barrier, 2)

        def issue_send(c):
            copy_in(c).wait()
            if c >= 2:
                x_rdma(c - 2).wait_send()
            a_bf16[c % 2, 0 : CHS[c], :] = a_f32[c % 2, 0 : CHS[c], :].astype(
                jnp.bfloat16
            )
            if c + 2 < NC:
                copy_in(c + 2).start()
            x_rdma(c).start()

        issue_send(0)
        issue_send(1)

        for c in range(NC):
            x_rdma(c).wait_recv()
            lo, hi = OFFS[c], OFFS[c] + CHS[c]
            xrecv[lo:hi, :] = xrecv[lo:hi, :] + a_bf16[c % 2, 0 : CHS[c], :]

            st_copy(c).start()
            y_rdma_send(c).start()
            if c + 2 < NC:
                issue_send(c + 2)

        for c in range(NC):
            y_rdma_recv(c).wait_recv()

        for c in (NC - 2, NC - 1):
            x_rdma(c).wait_send()
        for c in range(NC):
            y_rdma_send(c).wait_send()
            st_copy(c).wait()

    return pl.pallas_call(
        body,
        out_shape=jax.ShapeDtypeStruct((M, N), jnp.bfloat16),
        in_specs=[pl.BlockSpec(memory_space=pl.ANY)],
        out_specs=pl.BlockSpec(memory_space=pl.ANY),
        scratch_shapes=[
            pltpu.VMEM((2, CHMAX, N), jnp.float32),
            pltpu.VMEM((2, CHMAX, N), jnp.bfloat16),
            pltpu.VMEM((H, N), jnp.bfloat16),
            pltpu.SemaphoreType.DMA((NC,)),
            pltpu.SemaphoreType.DMA((NC,)),
            pltpu.SemaphoreType.DMA((NC,)),
            pltpu.SemaphoreType.DMA((NC,)),
            pltpu.SemaphoreType.DMA((NC,)),
            pltpu.SemaphoreType.DMA((NC,)),
        ],
        compiler_params=pltpu.CompilerParams(
            collective_id=0, vmem_limit_bytes=96 * 1024 * 1024
        ),
    )(x)


# device time: 418246 ns/iter; 1.0343x vs baseline; 1.0087x over previous
import jax
import jax.numpy as jnp
from jax import lax
from jax.experimental import pallas as pl
from jax.experimental.pallas import tpu as pltpu

M = 32768
N = 1024
H = M // 2

CHS = [256] * 64
assert sum(CHS) == H
NC = len(CHS)
OFFS = [sum(CHS[:c]) for c in range(NC)]
CHMAX = max(CHS)


def kernel(x):
    def body(
        x_ref,
        out_ref,
        a_f32,
        a_bf16,
        xrecv,
        in_sems,
        st_sems,
        xsend_sems,
        xrecv_sems,
        ysend_sems,
        yrecv_sems,
    ):
        my_x = lax.axis_index("x")
        my_y = lax.axis_index("y")
        x_nbr = (1 - my_x, my_y)
        y_nbr = (my_x, 1 - my_y)
        row0 = my_y * H
        other0 = (1 - my_y) * H

        def copy_in(c):
            return pltpu.make_async_copy(
                x_ref.at[pl.ds(row0 + OFFS[c], CHS[c])],
                a_f32.at[c % 2, pl.ds(0, CHS[c])],
                in_sems.at[c],
            )

        def x_rdma(c):
            return pltpu.make_async_remote_copy(
                src_ref=a_bf16.at[c % 2, pl.ds(0, CHS[c])],
                dst_ref=xrecv.at[pl.ds(OFFS[c], CHS[c])],
                send_sem=xsend_sems.at[c],
                recv_sem=xrecv_sems.at[c],
                device_id=x_nbr,
                device_id_type=pl.DeviceIdType.MESH,
            )

        def y_rdma_send(c):
            return pltpu.make_async_remote_copy(
                src_ref=xrecv.at[pl.ds(OFFS[c], CHS[c])],
                dst_ref=out_ref.at[pl.ds(row0 + OFFS[c], CHS[c])],
                send_sem=ysend_sems.at[c],
                recv_sem=yrecv_sems.at[c],
                device_id=y_nbr,
                device_id_type=pl.DeviceIdType.MESH,
            )

        def y_rdma_recv(c):
            return pltpu.make_async_remote_copy(
                src_ref=xrecv.at[pl.ds(OFFS[c], CHS[c])],
                dst_ref=out_ref.at[pl.ds(other0 + OFFS[c], CHS[c])],
                send_sem=ysend_sems.at[c],
                recv_sem=yrecv_sems.at[c],
                device_id=y_nbr,
                device_id_type=pl.DeviceIdType.MESH,
            )

        def st_copy(c):
            return pltpu.make_async_copy(
                xrecv.at[pl.ds(OFFS[c], CHS[c])],
                out_ref.at[pl.ds(row0 + OFFS[c], CHS[c])],
                st_sems.at[c],
            )

        copy_in(0).start()
        copy_in(1).start()

        barrier = pltpu.get_barrier_semaphore()
        for nbr in (x_nbr, y_nbr):
            pl.semaphore_signal(
                barrier, inc=1, device_id=nbr, device_id_type=pl.DeviceIdType.MESH
            )
        pl.semaphore_wait(barrier, 2)

        def issue_send(c):
            copy_in(c).wait()
            if c >= 2:
                x_rdma(c - 2).wait_send()
            a_bf16[c % 2, 0 : CHS[c], :] = a_f32[c % 2, 0 : CHS[c], :].astype(
                jnp.bfloat16
            )
            if c + 2 < NC:
                copy_in(c + 2).start()
            x_rdma(c).start()

        issue_send(0)
        issue_send(1)

        for c in range(NC):
            x_rdma(c).wait_recv()
            lo, hi = OFFS[c], OFFS[c] + CHS[c]
            xrecv[lo:hi, :] = xrecv[lo:hi, :] + a_bf16[c % 2, 0 : CHS[c], :]

            st_copy(c).start()
            y_rdma_send(c).start()
            if c + 2 < NC:
                issue_send(c + 2)

        for c in range(NC):
            y_rdma_recv(c).wait_recv()

        for c in (NC - 2, NC - 1):
            x_rdma(c).wait_send()
        for c in range(NC):
            y_rdma_send(c).wait_send()
            st_copy(c).wait()

    return pl.pallas_call(
        body,
        out_shape=jax.ShapeDtypeStruct((M, N), jnp.bfloat16),
        in_specs=[pl.BlockSpec(memory_space=pl.ANY)],
        out_specs=pl.BlockSpec(memory_space=pl.ANY),
        scratch_shapes=[
            pltpu.VMEM((2, CHMAX, N), jnp.float32),
            pltpu.VMEM((2, CHMAX, N), jnp.bfloat16),
            pltpu.VMEM((H, N), jnp.bfloat16),
            pltpu.SemaphoreType.DMA((NC,)),
            pltpu.SemaphoreType.DMA((NC,)),
            pltpu.SemaphoreType.DMA((NC,)),
            pltpu.SemaphoreType.DMA((NC,)),
            pltpu.SemaphoreType.DMA((NC,)),
            pltpu.SemaphoreType.DMA((NC,)),
        ],
        compiler_params=pltpu.CompilerParams(
            collective_id=0, vmem_limit_bytes=96 * 1024 * 1024
        ),
    )(x)
